# baseline (speedup 1.0000x reference)
import functools

import jax
import jax.numpy as jnp
from jax import lax
from jax.experimental import pallas as pl
from jax.experimental.pallas import tpu as pltpu
from jax.experimental.pallas import tpu_sc as plsc

KS = 5
N0 = 31360
E0 = 250880
N1 = 7840
E1 = 62720
NTILES = 32

_MESH = plsc.VectorSubcoreMesh(core_axis_name="c", subcore_axis_name="s")


E0_PER = E0 // NTILES
C1_CHUNKS = E0_PER // 16
A1_COLS = 26
A1_WORDS = N0 * A1_COLS
A1_TILE = A1_WORDS // 16
ZB1 = 5096


def _sc1_body(x_hbm, src_hbm, dst_hbm, pa_hbm, pb_hbm, z_hbm, out_hbm,
              xv, srcv, dstv, pav, pbv, istage0, istage1, vstage0, vstage1,
              zbuf, shared, ssem0, ssem1):
    cid = lax.axis_index("c")
    sid = lax.axis_index("s")
    base = (cid * 16 + sid) * E0_PER
    pltpu.sync_copy(x_hbm, xv)
    pltpu.sync_copy(src_hbm.at[pl.ds(base, E0_PER)], srcv)
    pltpu.sync_copy(dst_hbm.at[pl.ds(base, E0_PER)], dstv)
    pltpu.sync_copy(pa_hbm.at[pl.ds(base, E0_PER)], pav)
    pltpu.sync_copy(pb_hbm.at[pl.ds(base, E0_PER)], pbv)
    pltpu.sync_copy(z_hbm, zbuf)
    for j in range(10):
        pltpu.sync_copy(zbuf, shared.at[pl.ds(sid * A1_TILE + j * ZB1, ZB1)])
    plsc.subcore_barrier()

    ones = jnp.full((16,), 1.0, jnp.float32)
    istage = (istage0, istage1)
    vstage = (vstage0, vstage1)
    ssem = (ssem0, ssem1)

    def step(i2, carry):
        for b in range(2):
            i = i2 + b
            @pl.when(i2 >= 2)
            def _wait_scatter():
                pltpu.make_async_copy(vstage[b], shared.at[istage[b]],
                                      ssem[b]).wait()
            off = i * 16
            s = srcv[pl.ds(off, 16)]
            t = dstv[pl.ds(off, 16)]
            p0 = pav[pl.ds(off, 16)]
            p1 = pbv[pl.ds(off, 16)]
            xg = plsc.load_gather(xv, [s])
            pos0 = p0 * (KS - 1.0)
            pos1 = p1 * (KS - 1.0)
            lo0 = jnp.minimum(pos0.astype(jnp.int32), KS - 2)
            lo1 = jnp.minimum(pos1.astype(jnp.int32), KS - 2)
            f0 = pos0 - lo0.astype(jnp.float32)
            f1 = pos1 - lo1.astype(jnp.float32)
            g0 = 1.0 - f0
            g1 = 1.0 - f1
            drow = t * A1_COLS
            kbase = drow + lo0 * KS + lo1
            istage[b][pl.ds(0, 16)] = kbase
            istage[b][pl.ds(16, 16)] = kbase + 1
            istage[b][pl.ds(32, 16)] = kbase + KS
            istage[b][pl.ds(48, 16)] = kbase + KS + 1
            istage[b][pl.ds(64, 16)] = drow + 25
            vstage[b][pl.ds(0, 16)] = g0 * g1 * xg
            vstage[b][pl.ds(16, 16)] = g0 * f1 * xg
            vstage[b][pl.ds(32, 16)] = f0 * g1 * xg
            vstage[b][pl.ds(48, 16)] = f0 * f1 * xg
            vstage[b][pl.ds(64, 16)] = ones
            pltpu.async_copy(vstage[b], shared.at[istage[b]], ssem[b],
                             add=True)
        return carry

    lax.fori_loop(0, C1_CHUNKS // 2, lambda k, c: step(k * 2, c), 0)
    for b in range(2):
        pltpu.make_async_copy(vstage[b], shared.at[istage[b]], ssem[b]).wait()
    plsc.subcore_barrier()
    for j in range(10):
        off = sid * A1_TILE + j * ZB1
        pltpu.sync_copy(shared.at[pl.ds(off, ZB1)], zbuf)
        pltpu.sync_copy(zbuf, out_hbm.at[pl.ds(cid * A1_WORDS + off, ZB1)])


_SC_PARAMS = pltpu.CompilerParams(needs_layout_passes=False)

_sc1 = functools.partial(
    pl.kernel,
    out_type=jax.ShapeDtypeStruct((2 * A1_WORDS,), jnp.float32),
    mesh=_MESH,
    compiler_params=_SC_PARAMS,
    scratch_types=[
        pltpu.VMEM((N0,), jnp.float32),
        pltpu.VMEM((E0_PER,), jnp.int32),
        pltpu.VMEM((E0_PER,), jnp.int32),
        pltpu.VMEM((E0_PER,), jnp.float32),
        pltpu.VMEM((E0_PER,), jnp.float32),
        pltpu.VMEM((80,), jnp.int32),
        pltpu.VMEM((80,), jnp.int32),
        pltpu.VMEM((80,), jnp.float32),
        pltpu.VMEM((80,), jnp.float32),
        pltpu.VMEM((ZB1,), jnp.float32),
        pltpu.VMEM_SHARED((A1_WORDS,), jnp.float32),
        pltpu.SemaphoreType.DMA,
        pltpu.SemaphoreType.DMA,
    ],
)(_sc1_body)


PAIRS = 24
E1_PER = E1 // 16
C2_CHUNKS = E1_PER // 16
AGG_ROWS = 7936
AGG_TROWS = AGG_ROWS // 16
ZR2 = 8

_MESH1 = plsc.VectorSubcoreMesh(core_axis_name="c", subcore_axis_name="s",
                                num_cores=1)


def _sc2_idx(i, srcv, pav, pbv):
    off = i * 16
    s = srcv[pl.ds(off, 16)]
    p0 = pav[pl.ds(off, 16)]
    p1 = pbv[pl.ds(off, 16)]
    pos0 = p0 * (KS - 1.0)
    pos1 = p1 * (KS - 1.0)
    lo0 = jnp.minimum(pos0.astype(jnp.int32), KS - 2)
    lo1 = jnp.minimum(pos1.astype(jnp.int32), KS - 2)
    return s * PAIRS + lo0 * KS + lo1


def _sc2_coeffs(i, pav, pbv):
    off = i * 16
    p0 = pav[pl.ds(off, 16)]
    p1 = pbv[pl.ds(off, 16)]
    pos0 = p0 * (KS - 1.0)
    pos1 = p1 * (KS - 1.0)
    lo0 = jnp.minimum(pos0.astype(jnp.int32), KS - 2)
    lo1 = jnp.minimum(pos1.astype(jnp.int32), KS - 2)
    f0 = pos0 - lo0.astype(jnp.float32)
    f1 = pos1 - lo1.astype(jnp.float32)
    g0 = 1.0 - f0
    g1 = 1.0 - f1
    return g0 * g1, g0 * f1, f0 * g1, f0 * f1


def _sc2_body(xall_hbm, src_hbm, dst_hbm, pa_hbm, pb_hbm, z2_hbm, agg_hbm,
              srcv, dstv, pav, pbv,
              gstage0, gstage1, gstage2, gstage3,
              rows0, rows1, rows2, rows3,
              msg0, msg1, msg2, msg3,
              distage0, distage1, distage2, distage3, zrow2, agg_sh,
              gsem0, gsem1, gsem2, gsem3, ssem0, ssem1, ssem2, ssem3):
    sid = lax.axis_index("s")
    base = sid * E1_PER
    pltpu.sync_copy(src_hbm.at[pl.ds(base, E1_PER)], srcv)
    pltpu.sync_copy(dst_hbm.at[pl.ds(base, E1_PER)], dstv)
    pltpu.sync_copy(pa_hbm.at[pl.ds(base, E1_PER)], pav)
    pltpu.sync_copy(pb_hbm.at[pl.ds(base, E1_PER)], pbv)
    pltpu.sync_copy(z2_hbm, zrow2)
    for j in range(62):
        pltpu.sync_copy(zrow2, agg_sh.at[pl.ds(sid * AGG_TROWS + j * ZR2, ZR2)])
    plsc.subcore_barrier()

    NB = 4
    gstage = (gstage0, gstage1, gstage2, gstage3)
    rows = (rows0, rows1, rows2, rows3)
    msg = (msg0, msg1, msg2, msg3)
    distage = (distage0, distage1, distage2, distage3)
    gsem = (gsem0, gsem1, gsem2, gsem3)
    ssem = (ssem0, ssem1, ssem2, ssem3)

    e0 = jnp.where(lax.iota(jnp.int32, 16) == 0, 1.0, 0.0)
    zv = jnp.zeros((16,), jnp.float32)
    for b in range(NB):
        for e in range(16):
            msg[b][e, pl.ds(64, 16)] = e0
            msg[b][e, pl.ds(80, 16)] = zv
            msg[b][e, pl.ds(96, 16)] = zv
            msg[b][e, pl.ds(112, 16)] = zv

    def _stage_idx(b, i):
        gbase = _sc2_idx(i, srcv, pav, pbv)
        gstage[b][pl.ds(0, 16)] = gbase
        gstage[b][pl.ds(16, 16)] = gbase + KS

    for b in range(NB):
        _stage_idx(b, b)
        pltpu.async_copy(xall_hbm.at[gstage[b]], rows[b], gsem[b])

    def step(i2, carry):
        @pl.when(i2 >= NB)
        def _wait_scatters():
            for b in range(NB):
                pltpu.make_async_copy(msg[b], agg_sh.at[distage[b]],
                                      ssem[b]).wait()
        for b in range(NB):
            i = i2 + b
            pltpu.make_async_copy(xall_hbm.at[gstage[b]], rows[b],
                                  gsem[b]).wait()
            c00, c01, c10, c11 = _sc2_coeffs(i, pav, pbv)
            t = dstv[pl.ds(i * 16, 16)]
            for e in range(16):
                c0 = c00[e]
                c1 = c01[e]
                c2 = c10[e]
                c3 = c11[e]
                for cg in range(4):
                    sl = pl.ds(cg * 16, 16)
                    sh = pl.ds(64 + cg * 16, 16)
                    acc = ((c0 * rows[b][e, sl] + c1 * rows[b][e, sh])
                           + (c2 * rows[b][16 + e, sl]
                              + c3 * rows[b][16 + e, sh]))
                    msg[b][e, sl] = acc
            distage[b][...] = t
            pltpu.async_copy(msg[b], agg_sh.at[distage[b]], ssem[b],
                             add=True)
        @pl.when(i2 + 2 * NB <= C2_CHUNKS)
        def _prefetch():
            for b in range(NB):
                _stage_idx(b, i2 + b + NB)
                pltpu.async_copy(xall_hbm.at[gstage[b]], rows[b], gsem[b])
        return carry

    lax.fori_loop(0, C2_CHUNKS // NB, lambda k, c: step(k * NB, c), 0)
    i = C2_CHUNKS - 1
    _stage_idx(0, i)
    pltpu.async_copy(xall_hbm.at[gstage[0]], rows[0], gsem[0])
    pltpu.make_async_copy(xall_hbm.at[gstage[0]], rows[0], gsem[0]).wait()
    pltpu.make_async_copy(msg[0], agg_sh.at[distage[0]], ssem[0]).wait()
    c00, c01, c10, c11 = _sc2_coeffs(i, pav, pbv)
    t = dstv[pl.ds(i * 16, 16)]
    for e in range(16):
        c0 = c00[e]
        c1 = c01[e]
        c2 = c10[e]
        c3 = c11[e]
        for cg in range(4):
            sl = pl.ds(cg * 16, 16)
            sh = pl.ds(64 + cg * 16, 16)
            acc = (c0 * rows[0][e, sl] + c1 * rows[0][e, sh]
                   + c2 * rows[0][16 + e, sl] + c3 * rows[0][16 + e, sh])
            msg[0][e, sl] = acc
    distage[0][...] = t
    pltpu.async_copy(msg[0], agg_sh.at[distage[0]], ssem[0], add=True)
    for b in range(NB):
        pltpu.make_async_copy(msg[b], agg_sh.at[distage[b]], ssem[b]).wait()
    plsc.subcore_barrier()
    for j in range(62):
        r0 = sid * AGG_TROWS + j * ZR2
        pltpu.sync_copy(agg_sh.at[pl.ds(r0, ZR2)], zrow2)
        pltpu.sync_copy(zrow2, agg_hbm.at[pl.ds(r0, ZR2)])


_sc2 = functools.partial(
    pl.kernel,
    out_type=jax.ShapeDtypeStruct((AGG_ROWS, 128), jnp.float32),
    mesh=_MESH1,
    compiler_params=_SC_PARAMS,
    scratch_types=[
        pltpu.VMEM((E1_PER,), jnp.int32),
        pltpu.VMEM((E1_PER,), jnp.int32),
        pltpu.VMEM((E1_PER,), jnp.float32),
        pltpu.VMEM((E1_PER,), jnp.float32),
        pltpu.VMEM((32,), jnp.int32),
        pltpu.VMEM((32,), jnp.int32),
        pltpu.VMEM((32,), jnp.int32),
        pltpu.VMEM((32,), jnp.int32),
        pltpu.VMEM((32, 128), jnp.float32),
        pltpu.VMEM((32, 128), jnp.float32),
        pltpu.VMEM((32, 128), jnp.float32),
        pltpu.VMEM((32, 128), jnp.float32),
        pltpu.VMEM((16, 128), jnp.float32),
        pltpu.VMEM((16, 128), jnp.float32),
        pltpu.VMEM((16, 128), jnp.float32),
        pltpu.VMEM((16, 128), jnp.float32),
        pltpu.VMEM((16,), jnp.int32),
        pltpu.VMEM((16,), jnp.int32),
        pltpu.VMEM((16,), jnp.int32),
        pltpu.VMEM((16,), jnp.int32),
        pltpu.VMEM((ZR2, 128), jnp.float32),
        pltpu.VMEM_SHARED((AGG_ROWS, 128), jnp.float32),
        pltpu.SemaphoreType.DMA,
        pltpu.SemaphoreType.DMA,
        pltpu.SemaphoreType.DMA,
        pltpu.SemaphoreType.DMA,
        pltpu.SemaphoreType.DMA,
        pltpu.SemaphoreType.DMA,
        pltpu.SemaphoreType.DMA,
        pltpu.SemaphoreType.DMA,
    ],
)(_sc2_body)


def _elu(o):
    return jnp.where(o > 0, o, jnp.exp(o) - 1.0)


def _tcab_body(pr, xr, wr, rootr, br, w2r, h1r, xallr):
    a = pr[0] + pr[1]
    deg = jnp.maximum(a[:, 25:26], 1.0)
    agg = jnp.dot(a, wr[...], preferred_element_type=jnp.float32) / deg
    h = _elu(agg + xr[...] * rootr[...] + br[...])
    h4 = h.reshape(2, 14, 2, 14, 2, 32)
    h1 = jnp.max(jnp.max(h4, axis=4), axis=2).reshape(392, 32)
    h1r[...] = h1
    xallr[...] = jnp.dot(h1, w2r[...], preferred_element_type=jnp.float32)


def _tc_ab(a1p, x, w1pad, w1_root, b1, w2pair):
    bm = 1568
    return pl.pallas_call(
        _tcab_body,
        grid=(N0 // bm,),
        in_specs=[
            pl.BlockSpec((2, bm, A1_COLS), lambda m: (0, m, 0)),
            pl.BlockSpec((bm, 1), lambda m: (m, 0)),
            pl.BlockSpec((A1_COLS, 32), lambda m: (0, 0)),
            pl.BlockSpec((1, 32), lambda m: (0, 0)),
            pl.BlockSpec((1, 32), lambda m: (0, 0)),
            pl.BlockSpec((32, PAIRS * 128), lambda m: (0, 0)),
        ],
        out_specs=[
            pl.BlockSpec((392, 32), lambda m: (m, 0)),
            pl.BlockSpec((392, PAIRS * 128), lambda m: (m, 0)),
        ],
        out_shape=[
            jax.ShapeDtypeStruct((N1, 32), jnp.float32),
            jax.ShapeDtypeStruct((N1, PAIRS * 128), jnp.float32),
        ],
    )(a1p, x, w1pad, w1_root, b1, w2pair)


def _tccd_body(aggr, h1r, rootr, br, outr):
    a = aggr[...]
    deg = jnp.maximum(a[:, 64:65], 1.0)
    o = a[:, :64] / deg + jnp.dot(h1r[...], rootr[...],
                                  preferred_element_type=jnp.float32) + br[...]
    h2 = _elu(o)
    h4 = h2.reshape(2, 14, 2, 14, 2, 64)
    outr[...] = jnp.max(jnp.max(h4, axis=4), axis=2).reshape(392, 64)


def _tc_cd(aggp, h1, w2_root, b2):
    bm = 1568
    return pl.pallas_call(
        _tccd_body,
        grid=(N1 // bm,),
        in_specs=[
            pl.BlockSpec((bm, 128), lambda m: (m, 0)),
            pl.BlockSpec((bm, 32), lambda m: (m, 0)),
            pl.BlockSpec((32, 64), lambda m: (0, 0)),
            pl.BlockSpec((1, 64), lambda m: (0, 0)),
        ],
        out_specs=pl.BlockSpec((392, 64), lambda m: (m, 0)),
        out_shape=jax.ShapeDtypeStruct((1960, 64), jnp.float32),
    )(aggp, h1, w2_root, b2)


def _tce_body(xr, w1r, b1r, w2r, b2r, outr):
    z = _elu(jnp.dot(xr[...], w1r[...], preferred_element_type=jnp.float32)
             + b1r[...])
    z = _elu(jnp.dot(z, w2r[...], preferred_element_type=jnp.float32)
             + b2r[...])
    m = jnp.max(z, axis=-1, keepdims=True)
    s = jnp.sum(jnp.exp(z - m), axis=-1, keepdims=True)
    outr[...] = z - m - jnp.log(s)


def _tc_e(xf, fc1_w, fc1_b, fc2_w, fc2_b):
    return pl.pallas_call(
        _tce_body,
        out_shape=jax.ShapeDtypeStruct((10, 10), jnp.float32),
    )(xf, fc1_w, fc1_b, fc2_w, fc2_b)


def kernel(x, edge_index0, pseudo0, edge_index1, pseudo1, W1, W1_root, b1,
           W2, W2_root, b2, fc1_w, fc1_b, fc2_w, fc2_b):
    x = x.astype(jnp.float32)
    src0 = edge_index0[0].astype(jnp.int32)
    dst0 = edge_index0[1].astype(jnp.int32)
    zeros0 = jnp.zeros((ZB1,), jnp.float32)
    a1p = _sc1(x[:, 0], src0, dst0, pseudo0[:, 0], pseudo0[:, 1], zeros0)
    a1p = a1p.reshape(2, N0, A1_COLS)

    w1pad = jnp.pad(W1[:, 0, :], ((0, 1), (0, 0)))
    w2k = W2.transpose(1, 0, 2)
    w2pair = jnp.concatenate([w2k[:, :PAIRS, :], w2k[:, 1:PAIRS + 1, :]],
                             axis=-1).reshape(32, PAIRS * 128)
    h1, xall = _tc_ab(a1p, x, w1pad, W1_root, b1.reshape(1, 32), w2pair)

    src1 = edge_index1[0].astype(jnp.int32)
    dst1 = edge_index1[1].astype(jnp.int32)
    zeros2 = jnp.zeros((ZR2, 128), jnp.float32)
    aggp = _sc2(xall.reshape(N1 * PAIRS, 128), src1, dst1,
                pseudo1[:, 0], pseudo1[:, 1], zeros2)

    pooled = _tc_cd(aggp[:N1], h1, W2_root, b2.reshape(1, 64))
    xf = pooled.reshape(10, 196 * 64)
    return _tc_e(xf, fc1_w, fc1_b.reshape(1, 512), fc2_w, fc2_b.reshape(1, 10))


# speedup vs baseline: 1.1374x; 1.1374x over previous
import functools

import jax
import jax.numpy as jnp
from jax import lax
from jax.experimental import pallas as pl
from jax.experimental.pallas import tpu as pltpu
from jax.experimental.pallas import tpu_sc as plsc

KS = 5
N0 = 31360
E0 = 250880
N1 = 7840
E1 = 62720
NTILES = 32

_MESH = plsc.VectorSubcoreMesh(core_axis_name="c", subcore_axis_name="s")


E0_PER = E0 // NTILES
C1_CHUNKS = E0_PER // 16
A1_COLS = 26
A1_WORDS = N0 * A1_COLS
A1_TILE = A1_WORDS // 16
ZB1 = 5096


def _sc1_body(x_hbm, src_hbm, dst_hbm, pa_hbm, pb_hbm, z_hbm, out_hbm,
              xv, srcv, dstv, pav, pbv, istage0, istage1, vstage0, vstage1,
              zbuf, shared, ssem0, ssem1):
    cid = lax.axis_index("c")
    sid = lax.axis_index("s")
    base = (cid * 16 + sid) * E0_PER
    pltpu.sync_copy(x_hbm, xv)
    pltpu.sync_copy(src_hbm.at[pl.ds(base, E0_PER)], srcv)
    pltpu.sync_copy(dst_hbm.at[pl.ds(base, E0_PER)], dstv)
    pltpu.sync_copy(pa_hbm.at[pl.ds(base, E0_PER)], pav)
    pltpu.sync_copy(pb_hbm.at[pl.ds(base, E0_PER)], pbv)
    pltpu.sync_copy(z_hbm, zbuf)
    for j in range(10):
        pltpu.sync_copy(zbuf, shared.at[pl.ds(sid * A1_TILE + j * ZB1, ZB1)])
    plsc.subcore_barrier()

    ones = jnp.full((16,), 1.0, jnp.float32)
    istage = (istage0, istage1)
    vstage = (vstage0, vstage1)
    ssem = (ssem0, ssem1)

    def step(i2, carry):
        for b in range(2):
            i = i2 + b
            @pl.when(i2 >= 2)
            def _wait_scatter():
                pltpu.make_async_copy(vstage[b], shared.at[istage[b]],
                                      ssem[b]).wait()
            off = i * 16
            s = srcv[pl.ds(off, 16)]
            t = dstv[pl.ds(off, 16)]
            p0 = pav[pl.ds(off, 16)]
            p1 = pbv[pl.ds(off, 16)]
            xg = plsc.load_gather(xv, [s])
            pos0 = p0 * (KS - 1.0)
            pos1 = p1 * (KS - 1.0)
            lo0 = jnp.minimum(pos0.astype(jnp.int32), KS - 2)
            lo1 = jnp.minimum(pos1.astype(jnp.int32), KS - 2)
            f0 = pos0 - lo0.astype(jnp.float32)
            f1 = pos1 - lo1.astype(jnp.float32)
            g0 = 1.0 - f0
            g1 = 1.0 - f1
            drow = t * A1_COLS
            kbase = drow + lo0 * KS + lo1
            istage[b][pl.ds(0, 16)] = kbase
            istage[b][pl.ds(16, 16)] = kbase + 1
            istage[b][pl.ds(32, 16)] = kbase + KS
            istage[b][pl.ds(48, 16)] = kbase + KS + 1
            istage[b][pl.ds(64, 16)] = drow + 25
            vstage[b][pl.ds(0, 16)] = g0 * g1 * xg
            vstage[b][pl.ds(16, 16)] = g0 * f1 * xg
            vstage[b][pl.ds(32, 16)] = f0 * g1 * xg
            vstage[b][pl.ds(48, 16)] = f0 * f1 * xg
            vstage[b][pl.ds(64, 16)] = ones
            pltpu.async_copy(vstage[b], shared.at[istage[b]], ssem[b],
                             add=True)
        return carry

    lax.fori_loop(0, C1_CHUNKS // 2, lambda k, c: step(k * 2, c), 0)
    for b in range(2):
        pltpu.make_async_copy(vstage[b], shared.at[istage[b]], ssem[b]).wait()
    plsc.subcore_barrier()
    for j in range(10):
        off = sid * A1_TILE + j * ZB1
        pltpu.sync_copy(shared.at[pl.ds(off, ZB1)], zbuf)
        pltpu.sync_copy(zbuf, out_hbm.at[pl.ds(cid * A1_WORDS + off, ZB1)])


_SC_PARAMS = pltpu.CompilerParams(needs_layout_passes=False)

_sc1 = functools.partial(
    pl.kernel,
    out_type=jax.ShapeDtypeStruct((2 * A1_WORDS,), jnp.float32),
    mesh=_MESH,
    compiler_params=_SC_PARAMS,
    scratch_types=[
        pltpu.VMEM((N0,), jnp.float32),
        pltpu.VMEM((E0_PER,), jnp.int32),
        pltpu.VMEM((E0_PER,), jnp.int32),
        pltpu.VMEM((E0_PER,), jnp.float32),
        pltpu.VMEM((E0_PER,), jnp.float32),
        pltpu.VMEM((80,), jnp.int32),
        pltpu.VMEM((80,), jnp.int32),
        pltpu.VMEM((80,), jnp.float32),
        pltpu.VMEM((80,), jnp.float32),
        pltpu.VMEM((ZB1,), jnp.float32),
        pltpu.VMEM_SHARED((A1_WORDS,), jnp.float32),
        pltpu.SemaphoreType.DMA,
        pltpu.SemaphoreType.DMA,
    ],
)(_sc1_body)


PAIRS = 24
E1_PER = E1 // 16
C2_CHUNKS = E1_PER // 16
AGG_ROWS = 7936
AGG_TROWS = AGG_ROWS // 16
ZR2 = 8

_MESH1 = plsc.VectorSubcoreMesh(core_axis_name="c", subcore_axis_name="s",
                                num_cores=1)


def _sc2_idx(i, srcv, pav, pbv):
    off = i * 16
    s = srcv[pl.ds(off, 16)]
    p0 = pav[pl.ds(off, 16)]
    p1 = pbv[pl.ds(off, 16)]
    pos0 = p0 * (KS - 1.0)
    pos1 = p1 * (KS - 1.0)
    lo0 = jnp.minimum(pos0.astype(jnp.int32), KS - 2)
    lo1 = jnp.minimum(pos1.astype(jnp.int32), KS - 2)
    return s * PAIRS + lo0 * KS + lo1


def _sc2_coeffs(i, pav, pbv):
    off = i * 16
    p0 = pav[pl.ds(off, 16)]
    p1 = pbv[pl.ds(off, 16)]
    pos0 = p0 * (KS - 1.0)
    pos1 = p1 * (KS - 1.0)
    lo0 = jnp.minimum(pos0.astype(jnp.int32), KS - 2)
    lo1 = jnp.minimum(pos1.astype(jnp.int32), KS - 2)
    f0 = pos0 - lo0.astype(jnp.float32)
    f1 = pos1 - lo1.astype(jnp.float32)
    g0 = 1.0 - f0
    g1 = 1.0 - f1
    return g0 * g1, g0 * f1, f0 * g1, f0 * f1


def _sc2_body(xall_hbm, src_hbm, dst_hbm, pa_hbm, pb_hbm, z2_hbm, agg_hbm,
              srcv, dstv, pav, pbv,
              gstage0, gstage1, gstage2, gstage3,
              rows0, rows1, rows2, rows3,
              msg0, msg1, msg2, msg3,
              distage0, distage1, distage2, distage3, zrow2, agg_sh,
              gsem0, gsem1, gsem2, gsem3, ssem0, ssem1, ssem2, ssem3):
    sid = lax.axis_index("s")
    base = sid * E1_PER
    pltpu.sync_copy(src_hbm.at[pl.ds(base, E1_PER)], srcv)
    pltpu.sync_copy(dst_hbm.at[pl.ds(base, E1_PER)], dstv)
    pltpu.sync_copy(pa_hbm.at[pl.ds(base, E1_PER)], pav)
    pltpu.sync_copy(pb_hbm.at[pl.ds(base, E1_PER)], pbv)
    pltpu.sync_copy(z2_hbm, zrow2)
    for j in range(62):
        pltpu.sync_copy(zrow2, agg_sh.at[pl.ds(sid * AGG_TROWS + j * ZR2, ZR2)])
    plsc.subcore_barrier()

    NB = 4
    gstage = (gstage0, gstage1, gstage2, gstage3)
    rows = (rows0, rows1, rows2, rows3)
    msg = (msg0, msg1, msg2, msg3)
    distage = (distage0, distage1, distage2, distage3)
    gsem = (gsem0, gsem1, gsem2, gsem3)
    ssem = (ssem0, ssem1, ssem2, ssem3)

    e0 = jnp.where(lax.iota(jnp.int32, 16) == 0, 1.0, 0.0)
    zv = jnp.zeros((16,), jnp.float32)
    for b in range(NB):
        for e in range(16):
            msg[b][e, pl.ds(64, 16)] = e0
            msg[b][e, pl.ds(80, 16)] = zv
            msg[b][e, pl.ds(96, 16)] = zv
            msg[b][e, pl.ds(112, 16)] = zv

    def _stage_idx(b, i):
        gbase = _sc2_idx(i, srcv, pav, pbv)
        gstage[b][pl.ds(0, 16)] = gbase
        gstage[b][pl.ds(16, 16)] = gbase + KS

    for b in range(NB):
        _stage_idx(b, b)
        pltpu.async_copy(xall_hbm.at[gstage[b]], rows[b], gsem[b])

    def step(i2, carry):
        for b in range(NB):
            i = i2 + b
            pltpu.make_async_copy(xall_hbm.at[gstage[b]], rows[b],
                                  gsem[b]).wait()
            @pl.when(i2 >= NB)
            def _wait_scatter():
                pltpu.make_async_copy(msg[b], agg_sh.at[distage[b]],
                                      ssem[b]).wait()
            c00, c01, c10, c11 = _sc2_coeffs(i, pav, pbv)
            t = dstv[pl.ds(i * 16, 16)]
            for e in range(16):
                c0 = c00[e]
                c1 = c01[e]
                c2 = c10[e]
                c3 = c11[e]
                for cg in range(4):
                    sl = pl.ds(cg * 16, 16)
                    sh = pl.ds(64 + cg * 16, 16)
                    acc = ((c0 * rows[b][e, sl] + c1 * rows[b][e, sh])
                           + (c2 * rows[b][16 + e, sl]
                              + c3 * rows[b][16 + e, sh]))
                    msg[b][e, sl] = acc
            distage[b][...] = t
            pltpu.async_copy(msg[b], agg_sh.at[distage[b]], ssem[b],
                             add=True)
            @pl.when(i + NB < C2_CHUNKS)
            def _prefetch():
                _stage_idx(b, i + NB)
                pltpu.async_copy(xall_hbm.at[gstage[b]], rows[b], gsem[b])
        return carry

    lax.fori_loop(0, C2_CHUNKS // NB, lambda k, c: step(k * NB, c), 0)
    i = C2_CHUNKS - 1
    pltpu.make_async_copy(xall_hbm.at[gstage[0]], rows[0], gsem[0]).wait()
    pltpu.make_async_copy(msg[0], agg_sh.at[distage[0]], ssem[0]).wait()
    c00, c01, c10, c11 = _sc2_coeffs(i, pav, pbv)
    t = dstv[pl.ds(i * 16, 16)]
    for e in range(16):
        c0 = c00[e]
        c1 = c01[e]
        c2 = c10[e]
        c3 = c11[e]
        for cg in range(4):
            sl = pl.ds(cg * 16, 16)
            sh = pl.ds(64 + cg * 16, 16)
            acc = (c0 * rows[0][e, sl] + c1 * rows[0][e, sh]
                   + c2 * rows[0][16 + e, sl] + c3 * rows[0][16 + e, sh])
            msg[0][e, sl] = acc
    distage[0][...] = t
    pltpu.async_copy(msg[0], agg_sh.at[distage[0]], ssem[0], add=True)
    for b in range(NB):
        pltpu.make_async_copy(msg[b], agg_sh.at[distage[b]], ssem[b]).wait()
    plsc.subcore_barrier()
    for j in range(62):
        r0 = sid * AGG_TROWS + j * ZR2
        pltpu.sync_copy(agg_sh.at[pl.ds(r0, ZR2)], zrow2)
        pltpu.sync_copy(zrow2, agg_hbm.at[pl.ds(r0, ZR2)])


_sc2 = functools.partial(
    pl.kernel,
    out_type=jax.ShapeDtypeStruct((AGG_ROWS, 128), jnp.float32),
    mesh=_MESH1,
    compiler_params=_SC_PARAMS,
    scratch_types=[
        pltpu.VMEM((E1_PER,), jnp.int32),
        pltpu.VMEM((E1_PER,), jnp.int32),
        pltpu.VMEM((E1_PER,), jnp.float32),
        pltpu.VMEM((E1_PER,), jnp.float32),
        pltpu.VMEM((32,), jnp.int32),
        pltpu.VMEM((32,), jnp.int32),
        pltpu.VMEM((32,), jnp.int32),
        pltpu.VMEM((32,), jnp.int32),
        pltpu.VMEM((32, 128), jnp.float32),
        pltpu.VMEM((32, 128), jnp.float32),
        pltpu.VMEM((32, 128), jnp.float32),
        pltpu.VMEM((32, 128), jnp.float32),
        pltpu.VMEM((16, 128), jnp.float32),
        pltpu.VMEM((16, 128), jnp.float32),
        pltpu.VMEM((16, 128), jnp.float32),
        pltpu.VMEM((16, 128), jnp.float32),
        pltpu.VMEM((16,), jnp.int32),
        pltpu.VMEM((16,), jnp.int32),
        pltpu.VMEM((16,), jnp.int32),
        pltpu.VMEM((16,), jnp.int32),
        pltpu.VMEM((ZR2, 128), jnp.float32),
        pltpu.VMEM_SHARED((AGG_ROWS, 128), jnp.float32),
        pltpu.SemaphoreType.DMA,
        pltpu.SemaphoreType.DMA,
        pltpu.SemaphoreType.DMA,
        pltpu.SemaphoreType.DMA,
        pltpu.SemaphoreType.DMA,
        pltpu.SemaphoreType.DMA,
        pltpu.SemaphoreType.DMA,
        pltpu.SemaphoreType.DMA,
    ],
)(_sc2_body)


def _elu(o):
    return jnp.where(o > 0, o, jnp.exp(o) - 1.0)


def _tcab_body(pr, xr, wr, rootr, br, w2r, h1r, xallr):
    a = pr[0] + pr[1]
    deg = jnp.maximum(a[:, 25:26], 1.0)
    agg = jnp.dot(a, wr[...], preferred_element_type=jnp.float32) / deg
    h = _elu(agg + xr[...] * rootr[...] + br[...])
    h4 = h.reshape(2, 14, 2, 14, 2, 32)
    h1 = jnp.max(jnp.max(h4, axis=4), axis=2).reshape(392, 32)
    h1r[...] = h1
    xallr[...] = jnp.dot(h1, w2r[...], preferred_element_type=jnp.float32)


def _tc_ab(a1p, x, w1pad, w1_root, b1, w2pair):
    bm = 1568
    return pl.pallas_call(
        _tcab_body,
        grid=(N0 // bm,),
        in_specs=[
            pl.BlockSpec((2, bm, A1_COLS), lambda m: (0, m, 0)),
            pl.BlockSpec((bm, 1), lambda m: (m, 0)),
            pl.BlockSpec((A1_COLS, 32), lambda m: (0, 0)),
            pl.BlockSpec((1, 32), lambda m: (0, 0)),
            pl.BlockSpec((1, 32), lambda m: (0, 0)),
            pl.BlockSpec((32, PAIRS * 128), lambda m: (0, 0)),
        ],
        out_specs=[
            pl.BlockSpec((392, 32), lambda m: (m, 0)),
            pl.BlockSpec((392, PAIRS * 128), lambda m: (m, 0)),
        ],
        out_shape=[
            jax.ShapeDtypeStruct((N1, 32), jnp.float32),
            jax.ShapeDtypeStruct((N1, PAIRS * 128), jnp.float32),
        ],
    )(a1p, x, w1pad, w1_root, b1, w2pair)


def _tccd_body(aggr, h1r, rootr, br, outr):
    a = aggr[...]
    deg = jnp.maximum(a[:, 64:65], 1.0)
    o = a[:, :64] / deg + jnp.dot(h1r[...], rootr[...],
                                  preferred_element_type=jnp.float32) + br[...]
    h2 = _elu(o)
    h4 = h2.reshape(2, 14, 2, 14, 2, 64)
    outr[...] = jnp.max(jnp.max(h4, axis=4), axis=2).reshape(392, 64)


def _tc_cd(aggp, h1, w2_root, b2):
    bm = 1568
    return pl.pallas_call(
        _tccd_body,
        grid=(N1 // bm,),
        in_specs=[
            pl.BlockSpec((bm, 128), lambda m: (m, 0)),
            pl.BlockSpec((bm, 32), lambda m: (m, 0)),
            pl.BlockSpec((32, 64), lambda m: (0, 0)),
            pl.BlockSpec((1, 64), lambda m: (0, 0)),
        ],
        out_specs=pl.BlockSpec((392, 64), lambda m: (m, 0)),
        out_shape=jax.ShapeDtypeStruct((1960, 64), jnp.float32),
    )(aggp, h1, w2_root, b2)


def _tce_body(xr, w1r, b1r, w2r, b2r, outr):
    z = _elu(jnp.dot(xr[...], w1r[...], preferred_element_type=jnp.float32)
             + b1r[...])
    z = _elu(jnp.dot(z, w2r[...], preferred_element_type=jnp.float32)
             + b2r[...])
    m = jnp.max(z, axis=-1, keepdims=True)
    s = jnp.sum(jnp.exp(z - m), axis=-1, keepdims=True)
    outr[...] = z - m - jnp.log(s)


def _tc_e(xf, fc1_w, fc1_b, fc2_w, fc2_b):
    return pl.pallas_call(
        _tce_body,
        out_shape=jax.ShapeDtypeStruct((10, 10), jnp.float32),
    )(xf, fc1_w, fc1_b, fc2_w, fc2_b)


def kernel(x, edge_index0, pseudo0, edge_index1, pseudo1, W1, W1_root, b1,
           W2, W2_root, b2, fc1_w, fc1_b, fc2_w, fc2_b):
    x = x.astype(jnp.float32)
    src0 = edge_index0[0].astype(jnp.int32)
    dst0 = edge_index0[1].astype(jnp.int32)
    zeros0 = jnp.zeros((ZB1,), jnp.float32)
    a1p = _sc1(x[:, 0], src0, dst0, pseudo0[:, 0], pseudo0[:, 1], zeros0)
    a1p = a1p.reshape(2, N0, A1_COLS)

    w1pad = jnp.pad(W1[:, 0, :], ((0, 1), (0, 0)))
    w2k = W2.transpose(1, 0, 2)
    w2pair = jnp.concatenate([w2k[:, :PAIRS, :], w2k[:, 1:PAIRS + 1, :]],
                             axis=-1).reshape(32, PAIRS * 128)
    h1, xall = _tc_ab(a1p, x, w1pad, W1_root, b1.reshape(1, 32), w2pair)

    src1 = edge_index1[0].astype(jnp.int32)
    dst1 = edge_index1[1].astype(jnp.int32)
    zeros2 = jnp.zeros((ZR2, 128), jnp.float32)
    aggp = _sc2(xall.reshape(N1 * PAIRS, 128), src1, dst1,
                pseudo1[:, 0], pseudo1[:, 1], zeros2)

    pooled = _tc_cd(aggp[:N1], h1, W2_root, b2.reshape(1, 64))
    xf = pooled.reshape(10, 196 * 64)
    return _tc_e(xf, fc1_w, fc1_b.reshape(1, 512), fc2_w, fc2_b.reshape(1, 10))


# speedup vs baseline: 1.1467x; 1.0082x over previous
import functools

import jax
import jax.numpy as jnp
from jax import lax
from jax.experimental import pallas as pl
from jax.experimental.pallas import tpu as pltpu
from jax.experimental.pallas import tpu_sc as plsc

KS = 5
N0 = 31360
E0 = 250880
N1 = 7840
E1 = 62720
NTILES = 32

_MESH = plsc.VectorSubcoreMesh(core_axis_name="c", subcore_axis_name="s")


E0_PER = E0 // NTILES
C1_CHUNKS = E0_PER // 16
A1_COLS = 26
A1_WORDS = N0 * A1_COLS
A1_TILE = A1_WORDS // 16
ZB1 = 5096


def _sc1_body(x_hbm, src_hbm, dst_hbm, pa_hbm, pb_hbm, z_hbm, out_hbm,
              xv, srcv, dstv, pav, pbv, istage0, istage1, vstage0, vstage1,
              zbuf, shared, ssem0, ssem1):
    cid = lax.axis_index("c")
    sid = lax.axis_index("s")
    base = (cid * 16 + sid) * E0_PER
    pltpu.sync_copy(x_hbm, xv)
    pltpu.sync_copy(src_hbm.at[pl.ds(base, E0_PER)], srcv)
    pltpu.sync_copy(dst_hbm.at[pl.ds(base, E0_PER)], dstv)
    pltpu.sync_copy(pa_hbm.at[pl.ds(base, E0_PER)], pav)
    pltpu.sync_copy(pb_hbm.at[pl.ds(base, E0_PER)], pbv)
    pltpu.sync_copy(z_hbm, zbuf)
    for j in range(10):
        pltpu.sync_copy(zbuf, shared.at[pl.ds(sid * A1_TILE + j * ZB1, ZB1)])
    plsc.subcore_barrier()

    ones = jnp.full((16,), 1.0, jnp.float32)
    istage = (istage0, istage1)
    vstage = (vstage0, vstage1)
    ssem = (ssem0, ssem1)

    def step(i2, carry):
        for b in range(2):
            i = i2 + b
            @pl.when(i2 >= 2)
            def _wait_scatter():
                pltpu.make_async_copy(vstage[b], shared.at[istage[b]],
                                      ssem[b]).wait()
            off = i * 16
            s = srcv[pl.ds(off, 16)]
            t = dstv[pl.ds(off, 16)]
            p0 = pav[pl.ds(off, 16)]
            p1 = pbv[pl.ds(off, 16)]
            xg = plsc.load_gather(xv, [s])
            pos0 = p0 * (KS - 1.0)
            pos1 = p1 * (KS - 1.0)
            lo0 = jnp.minimum(pos0.astype(jnp.int32), KS - 2)
            lo1 = jnp.minimum(pos1.astype(jnp.int32), KS - 2)
            f0 = pos0 - lo0.astype(jnp.float32)
            f1 = pos1 - lo1.astype(jnp.float32)
            g0 = 1.0 - f0
            g1 = 1.0 - f1
            drow = t * A1_COLS
            kbase = drow + lo0 * KS + lo1
            istage[b][pl.ds(0, 16)] = kbase
            istage[b][pl.ds(16, 16)] = kbase + 1
            istage[b][pl.ds(32, 16)] = kbase + KS
            istage[b][pl.ds(48, 16)] = kbase + KS + 1
            istage[b][pl.ds(64, 16)] = drow + 25
            vstage[b][pl.ds(0, 16)] = g0 * g1 * xg
            vstage[b][pl.ds(16, 16)] = g0 * f1 * xg
            vstage[b][pl.ds(32, 16)] = f0 * g1 * xg
            vstage[b][pl.ds(48, 16)] = f0 * f1 * xg
            vstage[b][pl.ds(64, 16)] = ones
            pltpu.async_copy(vstage[b], shared.at[istage[b]], ssem[b],
                             add=True)
        return carry

    lax.fori_loop(0, C1_CHUNKS // 2, lambda k, c: step(k * 2, c), 0)
    for b in range(2):
        pltpu.make_async_copy(vstage[b], shared.at[istage[b]], ssem[b]).wait()
    plsc.subcore_barrier()
    for j in range(10):
        off = sid * A1_TILE + j * ZB1
        pltpu.sync_copy(shared.at[pl.ds(off, ZB1)], zbuf)
        pltpu.sync_copy(zbuf, out_hbm.at[pl.ds(cid * A1_WORDS + off, ZB1)])


_SC_PARAMS = pltpu.CompilerParams(needs_layout_passes=False)

_sc1 = functools.partial(
    pl.kernel,
    out_type=jax.ShapeDtypeStruct((2 * A1_WORDS,), jnp.float32),
    mesh=_MESH,
    compiler_params=_SC_PARAMS,
    scratch_types=[
        pltpu.VMEM((N0,), jnp.float32),
        pltpu.VMEM((E0_PER,), jnp.int32),
        pltpu.VMEM((E0_PER,), jnp.int32),
        pltpu.VMEM((E0_PER,), jnp.float32),
        pltpu.VMEM((E0_PER,), jnp.float32),
        pltpu.VMEM((80,), jnp.int32),
        pltpu.VMEM((80,), jnp.int32),
        pltpu.VMEM((80,), jnp.float32),
        pltpu.VMEM((80,), jnp.float32),
        pltpu.VMEM((ZB1,), jnp.float32),
        pltpu.VMEM_SHARED((A1_WORDS,), jnp.float32),
        pltpu.SemaphoreType.DMA,
        pltpu.SemaphoreType.DMA,
    ],
)(_sc1_body)


PAIRS = 24
E1_PER = E1 // 16
C2_CHUNKS = E1_PER // 16
AGG_ROWS = 7936
AGG_TROWS = AGG_ROWS // 16
ZR2 = 8

_MESH1 = plsc.VectorSubcoreMesh(core_axis_name="c", subcore_axis_name="s",
                                num_cores=1)


def _sc2_idx(i, srcv, pav, pbv):
    off = i * 16
    s = srcv[pl.ds(off, 16)]
    p0 = pav[pl.ds(off, 16)]
    p1 = pbv[pl.ds(off, 16)]
    pos0 = p0 * (KS - 1.0)
    pos1 = p1 * (KS - 1.0)
    lo0 = jnp.minimum(pos0.astype(jnp.int32), KS - 2)
    lo1 = jnp.minimum(pos1.astype(jnp.int32), KS - 2)
    return s * PAIRS + lo0 * KS + lo1


def _sc2_coeffs(i, pav, pbv):
    off = i * 16
    p0 = pav[pl.ds(off, 16)]
    p1 = pbv[pl.ds(off, 16)]
    pos0 = p0 * (KS - 1.0)
    pos1 = p1 * (KS - 1.0)
    lo0 = jnp.minimum(pos0.astype(jnp.int32), KS - 2)
    lo1 = jnp.minimum(pos1.astype(jnp.int32), KS - 2)
    f0 = pos0 - lo0.astype(jnp.float32)
    f1 = pos1 - lo1.astype(jnp.float32)
    g0 = 1.0 - f0
    g1 = 1.0 - f1
    return g0 * g1, g0 * f1, f0 * g1, f0 * f1


def _sc2_body(xall_hbm, src_hbm, dst_hbm, pa_hbm, pb_hbm, z2_hbm, agg_hbm,
              srcv, dstv, pav, pbv,
              gstage0, gstage1, gstage2, gstage3,
              rows0, rows1, rows2, rows3,
              msg0, msg1, msg2, msg3,
              distage0, distage1, distage2, distage3, zrow2, agg_sh,
              gsem0, gsem1, gsem2, gsem3, ssem0, ssem1, ssem2, ssem3):
    sid = lax.axis_index("s")
    base = sid * E1_PER
    pltpu.sync_copy(src_hbm.at[pl.ds(base, E1_PER)], srcv)
    pltpu.sync_copy(dst_hbm.at[pl.ds(base, E1_PER)], dstv)
    pltpu.sync_copy(pa_hbm.at[pl.ds(base, E1_PER)], pav)
    pltpu.sync_copy(pb_hbm.at[pl.ds(base, E1_PER)], pbv)
    pltpu.sync_copy(z2_hbm, zrow2)
    for j in range(62):
        pltpu.sync_copy(zrow2, agg_sh.at[pl.ds(sid * AGG_TROWS + j * ZR2, ZR2)])
    plsc.subcore_barrier()

    NB = 4
    gstage = (gstage0, gstage1, gstage2, gstage3)
    rows = (rows0, rows1, rows2, rows3)
    msg = (msg0, msg1, msg2, msg3)
    distage = (distage0, distage1, distage2, distage3)
    gsem = (gsem0, gsem1, gsem2, gsem3)
    ssem = (ssem0, ssem1, ssem2, ssem3)

    e0 = jnp.where(lax.iota(jnp.int32, 16) == 0, 1.0, 0.0)
    zv = jnp.zeros((16,), jnp.float32)
    for b in range(NB):
        for e in range(16):
            msg[b][e, pl.ds(64, 16)] = e0
            msg[b][e, pl.ds(80, 16)] = zv
            msg[b][e, pl.ds(96, 16)] = zv
            msg[b][e, pl.ds(112, 16)] = zv

    def _stage_idx(b, i):
        gbase = _sc2_idx(i, srcv, pav, pbv)
        gstage[b][pl.ds(0, 16)] = gbase
        gstage[b][pl.ds(16, 16)] = gbase + KS

    for b in range(NB):
        _stage_idx(b, b)
        pltpu.async_copy(xall_hbm.at[gstage[b]], rows[b], gsem[b])

    def step(i2, carry):
        for b in range(NB):
            i = i2 + b
            pltpu.make_async_copy(xall_hbm.at[gstage[b]], rows[b],
                                  gsem[b]).wait()
            @pl.when(i2 >= NB)
            def _wait_scatter():
                pltpu.make_async_copy(msg[b], agg_sh.at[distage[b]],
                                      ssem[b]).wait()
            c00, c01, c10, c11 = _sc2_coeffs(i, pav, pbv)
            t = dstv[pl.ds(i * 16, 16)]
            for e in range(16):
                c0 = c00[e]
                c1 = c01[e]
                c2 = c10[e]
                c3 = c11[e]
                for cg in range(4):
                    sl = pl.ds(cg * 16, 16)
                    sh = pl.ds(64 + cg * 16, 16)
                    acc = ((c0 * rows[b][e, sl] + c1 * rows[b][e, sh])
                           + (c2 * rows[b][16 + e, sl]
                              + c3 * rows[b][16 + e, sh]))
                    msg[b][e, sl] = acc
            distage[b][...] = t
            pltpu.async_copy(msg[b], agg_sh.at[distage[b]], ssem[b],
                             add=True)
            @pl.when(i + NB < C2_CHUNKS)
            def _prefetch():
                _stage_idx(b, i + NB)
                pltpu.async_copy(xall_hbm.at[gstage[b]], rows[b], gsem[b])
        return carry

    lax.fori_loop(0, C2_CHUNKS // NB, lambda k, c: step(k * NB, c), 0)
    i = C2_CHUNKS - 1
    pltpu.make_async_copy(xall_hbm.at[gstage[0]], rows[0], gsem[0]).wait()
    pltpu.make_async_copy(msg[0], agg_sh.at[distage[0]], ssem[0]).wait()
    c00, c01, c10, c11 = _sc2_coeffs(i, pav, pbv)
    t = dstv[pl.ds(i * 16, 16)]
    for e in range(16):
        c0 = c00[e]
        c1 = c01[e]
        c2 = c10[e]
        c3 = c11[e]
        for cg in range(4):
            sl = pl.ds(cg * 16, 16)
            sh = pl.ds(64 + cg * 16, 16)
            acc = (c0 * rows[0][e, sl] + c1 * rows[0][e, sh]
                   + c2 * rows[0][16 + e, sl] + c3 * rows[0][16 + e, sh])
            msg[0][e, sl] = acc
    distage[0][...] = t
    pltpu.async_copy(msg[0], agg_sh.at[distage[0]], ssem[0], add=True)
    for b in range(NB):
        pltpu.make_async_copy(msg[b], agg_sh.at[distage[b]], ssem[b]).wait()
    plsc.subcore_barrier()
    for j in range(62):
        r0 = sid * AGG_TROWS + j * ZR2
        pltpu.sync_copy(agg_sh.at[pl.ds(r0, ZR2)], zrow2)
        pltpu.sync_copy(zrow2, agg_hbm.at[pl.ds(r0, ZR2)])


_sc2 = functools.partial(
    pl.kernel,
    out_type=jax.ShapeDtypeStruct((AGG_ROWS, 128), jnp.float32),
    mesh=_MESH1,
    compiler_params=_SC_PARAMS,
    scratch_types=[
        pltpu.VMEM((E1_PER,), jnp.int32),
        pltpu.VMEM((E1_PER,), jnp.int32),
        pltpu.VMEM((E1_PER,), jnp.float32),
        pltpu.VMEM((E1_PER,), jnp.float32),
        pltpu.VMEM((32,), jnp.int32),
        pltpu.VMEM((32,), jnp.int32),
        pltpu.VMEM((32,), jnp.int32),
        pltpu.VMEM((32,), jnp.int32),
        pltpu.VMEM((32, 128), jnp.float32),
        pltpu.VMEM((32, 128), jnp.float32),
        pltpu.VMEM((32, 128), jnp.float32),
        pltpu.VMEM((32, 128), jnp.float32),
        pltpu.VMEM((16, 128), jnp.float32),
        pltpu.VMEM((16, 128), jnp.float32),
        pltpu.VMEM((16, 128), jnp.float32),
        pltpu.VMEM((16, 128), jnp.float32),
        pltpu.VMEM((16,), jnp.int32),
        pltpu.VMEM((16,), jnp.int32),
        pltpu.VMEM((16,), jnp.int32),
        pltpu.VMEM((16,), jnp.int32),
        pltpu.VMEM((ZR2, 128), jnp.float32),
        pltpu.VMEM_SHARED((AGG_ROWS, 128), jnp.float32),
        pltpu.SemaphoreType.DMA,
        pltpu.SemaphoreType.DMA,
        pltpu.SemaphoreType.DMA,
        pltpu.SemaphoreType.DMA,
        pltpu.SemaphoreType.DMA,
        pltpu.SemaphoreType.DMA,
        pltpu.SemaphoreType.DMA,
        pltpu.SemaphoreType.DMA,
    ],
)(_sc2_body)


def _elu(o):
    return jnp.where(o > 0, o, jnp.exp(o) - 1.0)


def _tcab_body(pr, xr, wr, rootr, br, w2r, h1r, xallr):
    a = pr[0] + pr[1]
    deg = jnp.maximum(a[:, 25:26], 1.0)
    agg = jnp.dot(a, wr[...], preferred_element_type=jnp.float32) / deg
    h = _elu(agg + xr[...] * rootr[...] + br[...])
    h4 = h.reshape(2, 14, 2, 14, 2, 32)
    h1 = jnp.max(jnp.max(h4, axis=4), axis=2).reshape(392, 32)
    h1r[...] = h1
    xallr[...] = jnp.dot(h1, w2r[...], preferred_element_type=jnp.float32)


def _tc_ab(a1p, x, w1pad, w1_root, b1, w2pair):
    bm = 1568
    return pl.pallas_call(
        _tcab_body,
        grid=(N0 // bm,),
        in_specs=[
            pl.BlockSpec((2, bm, A1_COLS), lambda m: (0, m, 0)),
            pl.BlockSpec((bm, 1), lambda m: (m, 0)),
            pl.BlockSpec((A1_COLS, 32), lambda m: (0, 0)),
            pl.BlockSpec((1, 32), lambda m: (0, 0)),
            pl.BlockSpec((1, 32), lambda m: (0, 0)),
            pl.BlockSpec((32, PAIRS * 128), lambda m: (0, 0)),
        ],
        out_specs=[
            pl.BlockSpec((392, 32), lambda m: (m, 0)),
            pl.BlockSpec((392, PAIRS * 128), lambda m: (m, 0)),
        ],
        out_shape=[
            jax.ShapeDtypeStruct((N1, 32), jnp.float32),
            jax.ShapeDtypeStruct((N1, PAIRS * 128), jnp.float32),
        ],
    )(a1p, x, w1pad, w1_root, b1, w2pair)


def _tccd_body(aggr, h1r, rootr, br, outr):
    a = aggr[...]
    deg = jnp.maximum(a[:, 64:65], 1.0)
    o = a[:, :64] / deg + jnp.dot(h1r[...], rootr[...],
                                  preferred_element_type=jnp.float32) + br[...]
    h2 = _elu(o)
    h4 = h2.reshape(2, 14, 2, 14, 2, 64)
    outr[...] = jnp.max(jnp.max(h4, axis=4), axis=2).reshape(392, 64)


def _tc_cd(aggp, h1, w2_root, b2):
    bm = 1568
    return pl.pallas_call(
        _tccd_body,
        grid=(N1 // bm,),
        in_specs=[
            pl.BlockSpec((bm, 128), lambda m: (m, 0)),
            pl.BlockSpec((bm, 32), lambda m: (m, 0)),
            pl.BlockSpec((32, 64), lambda m: (0, 0)),
            pl.BlockSpec((1, 64), lambda m: (0, 0)),
        ],
        out_specs=pl.BlockSpec((392, 64), lambda m: (m, 0)),
        out_shape=jax.ShapeDtypeStruct((1960, 64), jnp.float32),
    )(aggp, h1, w2_root, b2)


def _tce_body(xr, w1r, b1r, w2r, b2r, outr):
    z = _elu(jnp.dot(xr[...], w1r[...], preferred_element_type=jnp.float32)
             + b1r[...])
    z = _elu(jnp.dot(z, w2r[...], preferred_element_type=jnp.float32)
             + b2r[...])
    m = jnp.max(z, axis=-1, keepdims=True)
    s = jnp.sum(jnp.exp(z - m), axis=-1, keepdims=True)
    outr[...] = z - m - jnp.log(s)


def _tc_e(xf, fc1_w, fc1_b, fc2_w, fc2_b):
    return pl.pallas_call(
        _tce_body,
        out_shape=jax.ShapeDtypeStruct((10, 10), jnp.float32),
    )(xf, fc1_w, fc1_b, fc2_w, fc2_b)


def kernel(x, edge_index0, pseudo0, edge_index1, pseudo1, W1, W1_root, b1,
           W2, W2_root, b2, fc1_w, fc1_b, fc2_w, fc2_b):
    x = x.astype(jnp.float32)
    src0 = edge_index0[0].astype(jnp.int32)
    dst0 = edge_index0[1].astype(jnp.int32)
    zeros0 = jnp.zeros((ZB1,), jnp.float32)
    a1p = _sc1(x[:, 0], src0, dst0, pseudo0[:, 0], pseudo0[:, 1], zeros0)
    a1p = a1p.reshape(2, N0, A1_COLS)

    w1pad = jnp.pad(W1[:, 0, :], ((0, 1), (0, 0)))
    w2k = W2.transpose(1, 0, 2)
    w2pair = jnp.concatenate([w2k[:, :PAIRS, :], w2k[:, 1:PAIRS + 1, :]],
                             axis=-1).reshape(32, PAIRS * 128)
    h1, xall = _tc_ab(a1p, x, w1pad, W1_root, b1.reshape(1, 32), w2pair)

    src1 = edge_index1[0].astype(jnp.int32)
    dst1 = edge_index1[1].astype(jnp.int32)
    zeros2 = jnp.zeros((ZR2, 128), jnp.float32)
    aggp = _sc2(xall.reshape(N1 * PAIRS, 128), src1, dst1,
                pseudo1[:, 0], pseudo1[:, 1], zeros2)

    pooled = _tc_cd(aggp, h1, W2_root, b2.reshape(1, 64))
    xf = pooled.reshape(10, 196 * 64)
    return _tc_e(xf, fc1_w, fc1_b.reshape(1, 512), fc2_w, fc2_b.reshape(1, 10))
